# baseline (speedup 1.0000x reference)
import functools

import jax
import jax.numpy as jnp
from jax.experimental import pallas as pl
from jax.experimental.pallas import tpu as pltpu

_VMEM_BUDGET = 30 * 1024 * 1024
_VMEM_LIMIT_BYTES = 48 * 1024 * 1024


def _se_kernel(x_ref, w1_ref, w2_ref, o_ref, *, inv_hw):
    x = x_ref[...]
    pooled = jnp.sum(x, axis=(2, 3), dtype=jnp.float32) * inv_hw
    h = jnp.maximum(
        jnp.dot(pooled, w1_ref[...], preferred_element_type=jnp.float32), 0.0)
    gate = jax.nn.sigmoid(
        jnp.dot(h, w2_ref[...], preferred_element_type=jnp.float32))
    o_ref[...] = x * gate.astype(o_ref.dtype)[:, :, None, None]


def _pick_batch_tile(b, block_bytes_per_batch, budget_bytes):
    cap = 1 if b == 1 else b // 2
    best = 1
    for bt in range(1, cap + 1):
        if b % bt == 0 and 4 * bt * block_bytes_per_batch <= budget_bytes:
            best = bt
    return best


def kernel(x_nchw, w1, w2):
    B, C, H, W = x_nchw.shape
    HW = H * W
    Cr = w1.shape[1]

    itemsize = jnp.dtype(x_nchw.dtype).itemsize
    slab = (-(-H // 8) * 8) * (-(-W // 128) * 128) * itemsize
    bt = _pick_batch_tile(B, C * slab, _VMEM_BUDGET)

    body = functools.partial(_se_kernel, inv_hw=1.0 / float(HW))
    return pl.pallas_call(
        body,
        out_shape=jax.ShapeDtypeStruct((B, C, H, W), x_nchw.dtype),
        grid=(B // bt,),
        in_specs=[
            pl.BlockSpec((bt, C, H, W), lambda b: (b, 0, 0, 0)),
            pl.BlockSpec((C, Cr), lambda b: (0, 0)),
            pl.BlockSpec((Cr, C), lambda b: (0, 0)),
        ],
        out_specs=pl.BlockSpec((bt, C, H, W), lambda b: (b, 0, 0, 0)),
        compiler_params=pltpu.CompilerParams(
            dimension_semantics=("parallel",),
            vmem_limit_bytes=_VMEM_LIMIT_BYTES,
        ),
    )(x_nchw, w1, w2)


# speedup vs baseline: 12.4142x; 12.4142x over previous
import functools

import jax
import jax.numpy as jnp
from jax.experimental import pallas as pl
from jax.experimental.pallas import tpu as pltpu

_VMEM_BUDGET = 30 * 1024 * 1024
_VMEM_LIMIT_BYTES = 48 * 1024 * 1024


def _se_kernel(x_ref, w1_ref, w2_ref, o_ref, *, inv_hw):
    x = x_ref[...]
    pooled = jnp.sum(x, axis=0, dtype=jnp.float32) * inv_hw
    h = jnp.maximum(
        jnp.dot(pooled, w1_ref[...], preferred_element_type=jnp.float32), 0.0)
    gate = jax.nn.sigmoid(
        jnp.dot(h, w2_ref[...], preferred_element_type=jnp.float32))
    o_ref[...] = x * gate.astype(o_ref.dtype)[None]


def _pick_batch_tile(b, bytes_per_batch, budget_bytes):
    cap = 1 if b == 1 else b // 2
    best = 1
    for bt in range(1, cap + 1):
        if b % bt == 0 and 4 * bt * bytes_per_batch <= budget_bytes:
            best = bt
    return best


def kernel(x_nchw, w1, w2):
    B, C, H, W = x_nchw.shape
    HW = H * W
    Cr = w1.shape[1]

    xt = jnp.transpose(x_nchw, (2, 3, 0, 1)).reshape(HW, B, C)

    itemsize = jnp.dtype(x_nchw.dtype).itemsize
    bt = _pick_batch_tile(B, HW * C * itemsize, _VMEM_BUDGET)

    body = functools.partial(_se_kernel, inv_hw=1.0 / float(HW))
    out = pl.pallas_call(
        body,
        out_shape=jax.ShapeDtypeStruct((HW, B, C), x_nchw.dtype),
        grid=(B // bt,),
        in_specs=[
            pl.BlockSpec((HW, bt, C), lambda b: (0, b, 0)),
            pl.BlockSpec((C, Cr), lambda b: (0, 0)),
            pl.BlockSpec((Cr, C), lambda b: (0, 0)),
        ],
        out_specs=pl.BlockSpec((HW, bt, C), lambda b: (0, b, 0)),
        compiler_params=pltpu.CompilerParams(
            dimension_semantics=("parallel",),
            vmem_limit_bytes=_VMEM_LIMIT_BYTES,
        ),
    )(xt, w1, w2)

    return out.reshape(H, W, B, C).transpose(2, 3, 0, 1)


# speedup vs baseline: 13.0084x; 1.0479x over previous
import functools

import jax
import jax.numpy as jnp
from jax.experimental import pallas as pl
from jax.experimental.pallas import tpu as pltpu

_VMEM_BUDGET = 54 * 1024 * 1024
_VMEM_LIMIT_BYTES = 58 * 1024 * 1024


def _se_kernel(x_ref, w1_ref, w2_ref, o_ref, *, inv_hw):
    x = x_ref[...]
    pooled = jnp.sum(x, axis=0, dtype=jnp.float32) * inv_hw
    h = jnp.maximum(
        jnp.dot(pooled, w1_ref[...], preferred_element_type=jnp.float32), 0.0)
    gate = jax.nn.sigmoid(
        jnp.dot(h, w2_ref[...], preferred_element_type=jnp.float32))
    o_ref[...] = x * gate.astype(o_ref.dtype)[None]


def _pick_batch_tile(b, bytes_per_batch, budget_bytes):
    cap = 1 if b == 1 else b // 2
    best = 1
    for bt in range(1, cap + 1):
        if b % bt == 0 and 4 * bt * bytes_per_batch <= budget_bytes:
            best = bt
    return best


def kernel(x_nchw, w1, w2):
    B, C, H, W = x_nchw.shape
    HW = H * W
    Cr = w1.shape[1]

    xt = jnp.transpose(x_nchw, (2, 3, 0, 1)).reshape(HW, B, C)

    itemsize = jnp.dtype(x_nchw.dtype).itemsize
    bt = _pick_batch_tile(B, HW * C * itemsize, _VMEM_BUDGET)

    body = functools.partial(_se_kernel, inv_hw=1.0 / float(HW))
    out = pl.pallas_call(
        body,
        out_shape=jax.ShapeDtypeStruct((HW, B, C), x_nchw.dtype),
        grid=(B // bt,),
        in_specs=[
            pl.BlockSpec((HW, bt, C), lambda b: (0, b, 0)),
            pl.BlockSpec((C, Cr), lambda b: (0, 0)),
            pl.BlockSpec((Cr, C), lambda b: (0, 0)),
        ],
        out_specs=pl.BlockSpec((HW, bt, C), lambda b: (0, b, 0)),
        compiler_params=pltpu.CompilerParams(
            dimension_semantics=("parallel",),
            vmem_limit_bytes=_VMEM_LIMIT_BYTES,
        ),
    )(xt, w1, w2)

    return out.reshape(H, W, B, C).transpose(2, 3, 0, 1)


# speedup vs baseline: 13.2871x; 1.0214x over previous
import functools

import jax
import jax.numpy as jnp
from jax.experimental import pallas as pl
from jax.experimental.pallas import tpu as pltpu

_VMEM_BUDGET = 54 * 1024 * 1024
_VMEM_LIMIT_BYTES = 58 * 1024 * 1024


def _se_kernel(x_ref, w1t_ref, w2_ref, o_ref, *, inv_hw):
    x = x_ref[...]
    pooled = jnp.sum(x, axis=0, dtype=jnp.float32) * inv_hw
    h = jnp.maximum(
        jax.lax.dot_general(pooled, w1t_ref[...],
                            (((1,), (1,)), ((), ())),
                            preferred_element_type=jnp.float32), 0.0)
    gate = jax.nn.sigmoid(
        jnp.dot(h, w2_ref[...], preferred_element_type=jnp.float32))
    o_ref[...] = x * gate.astype(o_ref.dtype)[None]


def _pick_batch_tile(b, bytes_per_batch, budget_bytes):
    cap = 1 if b == 1 else b // 2
    best = 1
    for bt in range(1, cap + 1):
        if b % bt == 0 and 4 * bt * bytes_per_batch <= budget_bytes:
            best = bt
    return best


def kernel(x_nchw, w1, w2):
    B, C, H, W = x_nchw.shape
    HW = H * W
    Cr = w1.shape[1]

    xt = jnp.transpose(x_nchw, (2, 3, 0, 1)).reshape(HW, B, C)
    w1t = jnp.transpose(w1)

    itemsize = jnp.dtype(x_nchw.dtype).itemsize
    bt = _pick_batch_tile(B, HW * C * itemsize, _VMEM_BUDGET)

    body = functools.partial(_se_kernel, inv_hw=1.0 / float(HW))
    out = pl.pallas_call(
        body,
        out_shape=jax.ShapeDtypeStruct((HW, B, C), x_nchw.dtype),
        grid=(B // bt,),
        in_specs=[
            pl.BlockSpec((HW, bt, C), lambda b: (0, b, 0)),
            pl.BlockSpec((Cr, C), lambda b: (0, 0)),
            pl.BlockSpec((Cr, C), lambda b: (0, 0)),
        ],
        out_specs=pl.BlockSpec((HW, bt, C), lambda b: (0, b, 0)),
        compiler_params=pltpu.CompilerParams(
            dimension_semantics=("parallel",),
            vmem_limit_bytes=_VMEM_LIMIT_BYTES,
        ),
    )(xt, w1t, w2)

    return out.reshape(H, W, B, C).transpose(2, 3, 0, 1)
